# baseline (speedup 1.0000x reference)
import functools

import jax
import jax.numpy as jnp
from jax import lax
from jax.experimental import pallas as pl
from jax.experimental.pallas import tpu as pltpu
from jax.experimental.pallas import tpu_sc as plsc

_CHUNK = 128
_GROUP = 4


@functools.lru_cache(maxsize=None)
def _build(n_rows, d, n_chunks, nc, ns):
    nw = nc * ns
    n_chunks_w = n_chunks // nw
    n_groups = n_chunks_w // _GROUP
    mesh = plsc.VectorSubcoreMesh(
        core_axis_name="c", subcore_axis_name="s",
        num_cores=nc, num_subcores=ns)

    @functools.partial(
        pl.kernel,
        out_type=jax.ShapeDtypeStruct((n_chunks, _CHUNK, d), jnp.float32),
        mesh=mesh,
        compiler_params=pltpu.CompilerParams(use_tc_tiling_on_sc=False),
        scratch_types=[
            pltpu.VMEM((n_chunks_w, _CHUNK), jnp.int32),
            pltpu.VMEM((2, _GROUP, _CHUNK, d), jnp.float32),
            pltpu.SemaphoreType.DMA,
        ],
    )
    def gather_kernel(idx_hbm, table_hbm, out_hbm, idx_v, rows_v, gsem):
        wid = lax.axis_index("s") * nc + lax.axis_index("c")
        chunk_base = wid * n_chunks_w
        pltpu.sync_copy(idx_hbm.at[pl.ds(chunk_base, n_chunks_w)], idx_v)

        def issue(g, half):
            for b in range(_GROUP):
                pltpu.async_copy(
                    table_hbm.at[idx_v.at[g * _GROUP + b]],
                    rows_v.at[half].at[b], gsem)

        def drain(half):
            for b in range(_GROUP):
                pltpu.make_async_copy(
                    table_hbm.at[pl.ds(0, _CHUNK)],
                    rows_v.at[half].at[b], gsem).wait()

        def store(g, half):
            pltpu.sync_copy(
                rows_v.at[half],
                out_hbm.at[pl.ds(chunk_base + g * _GROUP, _GROUP)])

        issue(0, 0)

        @pl.loop(0, n_groups - 2, step=2)
        def _(g):
            issue(g + 1, 1)
            drain(0)
            store(g, 0)
            issue(g + 2, 0)
            drain(1)
            store(g + 1, 1)

        issue(n_groups - 1, 1)
        drain(0)
        store(n_groups - 2, 0)
        drain(1)
        store(n_groups - 1, 1)

    return gather_kernel


def kernel(x, table):
    batch, fields = x.shape
    vocab, d = table.shape
    b_total = batch * fields
    idx = x.reshape(b_total).astype(jnp.int32).reshape(b_total // _CHUNK, _CHUNK)
    fn = _build(b_total, d, b_total // _CHUNK, 2, 16)
    out = fn(idx, table)
    return out.reshape(batch, fields, d)


# speedup vs baseline: 1.0234x; 1.0234x over previous
import functools

import jax
import jax.numpy as jnp
from jax import lax
from jax.experimental import pallas as pl
from jax.experimental.pallas import tpu as pltpu
from jax.experimental.pallas import tpu_sc as plsc

_CHUNK = 128
_GROUP = 2


@functools.lru_cache(maxsize=None)
def _build(n_rows, d, n_chunks, nc, ns):
    nw = nc * ns
    n_chunks_w = n_chunks // nw
    n_groups = n_chunks_w // _GROUP
    mesh = plsc.VectorSubcoreMesh(
        core_axis_name="c", subcore_axis_name="s",
        num_cores=nc, num_subcores=ns)

    @functools.partial(
        pl.kernel,
        out_type=jax.ShapeDtypeStruct((n_chunks, _CHUNK, d), jnp.float32),
        mesh=mesh,
        compiler_params=pltpu.CompilerParams(use_tc_tiling_on_sc=False),
        scratch_types=[
            pltpu.VMEM((n_chunks_w, _CHUNK), jnp.int32),
            pltpu.VMEM((2, _GROUP, _CHUNK, 128), jnp.float32),
            pltpu.SemaphoreType.DMA,
        ],
    )
    def gather_kernel(idx_hbm, table_hbm, out_hbm, idx_v, rows_v, gsem):
        wid = lax.axis_index("s") * nc + lax.axis_index("c")
        chunk_base = wid * n_chunks_w
        pltpu.sync_copy(idx_hbm.at[pl.ds(chunk_base, n_chunks_w)], idx_v)

        def issue(g, half):
            for b in range(_GROUP):
                pltpu.async_copy(
                    table_hbm.at[idx_v.at[g * _GROUP + b]],
                    rows_v.at[half].at[b], gsem)

        def drain(half):
            for b in range(_GROUP):
                pltpu.make_async_copy(
                    table_hbm.at[pl.ds(0, _CHUNK)],
                    rows_v.at[half].at[b], gsem).wait()

        def store(g, half):
            pltpu.sync_copy(
                rows_v.at[half, :, :, pl.ds(0, d)],
                out_hbm.at[pl.ds(chunk_base + g * _GROUP, _GROUP)])

        issue(0, 0)

        @pl.loop(0, n_groups - 2, step=2)
        def _(g):
            issue(g + 1, 1)
            drain(0)
            store(g, 0)
            issue(g + 2, 0)
            drain(1)
            store(g + 1, 1)

        issue(n_groups - 1, 1)
        drain(0)
        store(n_groups - 2, 0)
        drain(1)
        store(n_groups - 1, 1)

    return gather_kernel


def kernel(x, table):
    batch, fields = x.shape
    vocab, d = table.shape
    b_total = batch * fields
    idx = x.reshape(b_total).astype(jnp.int32).reshape(b_total // _CHUNK, _CHUNK)
    tablep = jnp.concatenate(
        [table, jnp.zeros((vocab, 128 - d), jnp.float32)], axis=1)
    fn = _build(b_total, d, b_total // _CHUNK, 2, 16)
    out = fn(idx, tablep)
    return out.reshape(batch, fields, d)


# speedup vs baseline: 1.0721x; 1.0475x over previous
import functools

import jax
import jax.numpy as jnp
from jax import lax
from jax.experimental import pallas as pl
from jax.experimental.pallas import tpu as pltpu
from jax.experimental.pallas import tpu_sc as plsc

_CHUNK = 128
_GROUP = 4


@functools.lru_cache(maxsize=None)
def _build(n_rows, d, n_chunks, nc, ns):
    nw = nc * ns
    n_chunks_w = n_chunks // nw
    n_groups = n_chunks_w // _GROUP
    mesh = plsc.VectorSubcoreMesh(
        core_axis_name="c", subcore_axis_name="s",
        num_cores=nc, num_subcores=ns)

    @functools.partial(
        pl.kernel,
        out_type=jax.ShapeDtypeStruct((n_chunks, _CHUNK, d), jnp.float32),
        mesh=mesh,
        compiler_params=pltpu.CompilerParams(use_tc_tiling_on_sc=False),
        scratch_types=[
            pltpu.VMEM((n_chunks_w, _CHUNK), jnp.int32),
            pltpu.VMEM((2, _GROUP, _CHUNK, d), jnp.float32),
            pltpu.SemaphoreType.DMA,
        ],
    )
    def gather_kernel(idx_hbm, table_hbm, out_hbm, idx_v, rows_v, gsem):
        wid = lax.axis_index("s") * nc + lax.axis_index("c")
        chunk_base = wid * n_chunks_w
        pltpu.sync_copy(idx_hbm.at[pl.ds(chunk_base, n_chunks_w)], idx_v)

        def issue(g, half):
            for b in range(_GROUP):
                pltpu.async_copy(
                    table_hbm.at[idx_v.at[g * _GROUP + b]],
                    rows_v.at[half].at[b], gsem)

        def drain(half):
            for b in range(_GROUP):
                pltpu.make_async_copy(
                    table_hbm.at[pl.ds(0, _CHUNK)],
                    rows_v.at[half].at[b], gsem).wait()

        def store(g, half):
            pltpu.sync_copy(
                rows_v.at[half],
                out_hbm.at[pl.ds(chunk_base + g * _GROUP, _GROUP)])

        issue(0, 0)

        @pl.loop(0, n_groups - 2, step=2)
        def _(g):
            issue(g + 1, 1)
            drain(0)
            store(g, 0)
            issue(g + 2, 0)
            drain(1)
            store(g + 1, 1)

        issue(n_groups - 1, 1)
        drain(0)
        store(n_groups - 2, 0)
        drain(1)
        store(n_groups - 1, 1)

    return gather_kernel


def kernel(x, table):
    batch, fields = x.shape
    vocab, d = table.shape
    b_total = batch * fields
    idx = (x.reshape(b_total).astype(jnp.int32) * 2).reshape(
        b_total // _CHUNK, _CHUNK)
    tablep = jnp.concatenate(
        [table, jnp.zeros((vocab, 128 - d), jnp.float32)], axis=1)
    table2 = tablep.reshape(2 * vocab, d)
    fn = _build(b_total, d, b_total // _CHUNK, 2, 16)
    out = fn(idx, table2)
    return out.reshape(batch, fields, d)
